# Initial kernel scaffold; baseline (speedup 1.0000x reference)
import jax
import jax.numpy as jnp
from jax.experimental import pallas as pl


def kernel(x, edge_index, W_self1, W_neigh1, b1, Wih, Whh, bih, bhh, W_self2, W_neigh2, b2):
    raise NotImplementedError("write your pallas kernel here")


# speedup vs baseline: 3.3042x; 3.3042x over previous
import functools

import jax
import jax.numpy as jnp
from jax import lax
from jax.experimental import pallas as pl
from jax.experimental.pallas import tpu as pltpu
from jax.experimental.pallas import tpu_sc as plsc

_NUM_CORES = 2
_NUM_SUBCORES = 16
_NW = _NUM_CORES * _NUM_SUBCORES
_LANES = 16


def _sc_mean(x, src_pad, deg, npad, per_w, ch_nodes):
    n, d = x.shape
    rows = ch_nodes * deg
    mesh = plsc.VectorSubcoreMesh(core_axis_name="c", subcore_axis_name="s")

    @functools.partial(
        pl.kernel,
        mesh=mesh,
        out_type=jax.ShapeDtypeStruct((npad, d), jnp.float32),
        scratch_types=[
            pltpu.VMEM((rows,), jnp.int32),
            pltpu.VMEM((rows, d), jnp.float32),
            pltpu.VMEM((ch_nodes, d), jnp.float32),
            pltpu.SemaphoreType.DMA,
        ],
    )
    def meank(x_hbm, src_hbm, out_hbm, idx_v, rows_v, acc_v, sem):
        wid = lax.axis_index("s") * _NUM_CORES + lax.axis_index("c")
        scale = jnp.float32(1.0 / deg)

        def chunk(ci, carry):
            n0 = wid * per_w + ci * ch_nodes
            pltpu.sync_copy(src_hbm.at[pl.ds(n0 * deg, rows)], idx_v)
            pltpu.async_copy(x_hbm.at[idx_v], rows_v, sem).wait()

            def node_body(j, c2):
                def col_body(c, c3):
                    acc = rows_v[j * deg, pl.ds(c * _LANES, _LANES)]
                    for k in range(1, deg):
                        acc = acc + rows_v[j * deg + k, pl.ds(c * _LANES, _LANES)]
                    acc_v[j, pl.ds(c * _LANES, _LANES)] = acc * scale
                    return c3

                return lax.fori_loop(0, d // _LANES, col_body, c2)

            lax.fori_loop(0, ch_nodes, node_body, 0)
            pltpu.sync_copy(acc_v, out_hbm.at[pl.ds(n0, ch_nodes)])
            return carry

        lax.fori_loop(0, per_w // ch_nodes, chunk, 0)

    return meank(x, src_pad)


def _sc_gather(table, idx, per_w, ch):
    e = idx.shape[0]
    d = table.shape[1]
    mesh = plsc.VectorSubcoreMesh(core_axis_name="c", subcore_axis_name="s")

    @functools.partial(
        pl.kernel,
        mesh=mesh,
        out_type=jax.ShapeDtypeStruct((e, d), jnp.float32),
        scratch_types=[
            pltpu.VMEM((ch,), jnp.int32),
            pltpu.VMEM((ch, d), jnp.float32),
            pltpu.SemaphoreType.DMA,
        ],
    )
    def gatherk(tab_hbm, idx_hbm, out_hbm, idx_v, rows_v, sem):
        wid = lax.axis_index("s") * _NUM_CORES + lax.axis_index("c")

        def chunk(ci, carry):
            e0 = wid * per_w + ci * ch
            pltpu.sync_copy(idx_hbm.at[pl.ds(e0, ch)], idx_v)
            pltpu.async_copy(tab_hbm.at[idx_v], rows_v, sem).wait()
            pltpu.sync_copy(rows_v, out_hbm.at[pl.ds(e0, ch)])
            return carry

        lax.fori_loop(0, per_w // ch, chunk, 0)

    return gatherk(table, idx)


def _tc_layer1(x, mean_neigh, w_self, w_neigh, b, bn):
    n, d = x.shape
    h = w_self.shape[1]

    def body(x_ref, m_ref, ws_ref, wn_ref, b_ref, o_ref):
        s = jnp.dot(x_ref[...], ws_ref[...], preferred_element_type=jnp.float32)
        s = s + jnp.dot(m_ref[...], wn_ref[...], preferred_element_type=jnp.float32)
        s = s + b_ref[...]
        o_ref[...] = jnp.where(s > 0, s, jnp.exp(jnp.minimum(s, 0.0)) - 1.0)

    return pl.pallas_call(
        body,
        grid=(n // bn,),
        in_specs=[
            pl.BlockSpec((bn, d), lambda i: (i, 0)),
            pl.BlockSpec((bn, d), lambda i: (i, 0)),
            pl.BlockSpec((d, h), lambda i: (0, 0)),
            pl.BlockSpec((d, h), lambda i: (0, 0)),
            pl.BlockSpec((1, h), lambda i: (0, 0)),
        ],
        out_specs=pl.BlockSpec((bn, h), lambda i: (i, 0)),
        out_shape=jax.ShapeDtypeStruct((n, h), jnp.float32),
        compiler_params=pltpu.CompilerParams(dimension_semantics=("parallel",)),
    )(x, mean_neigh, w_self, w_neigh, b.reshape(1, h))


def _tc_lstm_out(seq, h, wih, whh, bg, ws2, wn2, b2, bn):
    deg, n, hd = seq.shape
    h4 = wih.shape[1]
    c_out = ws2.shape[1]

    def body(seq_ref, h_ref, wih_ref, whh_ref, bg_ref, ws2_ref, wn2_ref,
             b2_ref, o_ref, hp, cp):
        t = pl.program_id(1)

        @pl.when(t == 0)
        def _():
            hp[...] = jnp.zeros_like(hp)
            cp[...] = jnp.zeros_like(cp)

        xt = seq_ref[0]
        gates = jnp.dot(xt, wih_ref[...], preferred_element_type=jnp.float32)
        gates = gates + jnp.dot(hp[...], whh_ref[...],
                                preferred_element_type=jnp.float32)
        gates = gates + bg_ref[...]
        i_g = jax.nn.sigmoid(gates[:, :hd])
        f_g = jax.nn.sigmoid(gates[:, hd:2 * hd])
        g_g = jnp.tanh(gates[:, 2 * hd:3 * hd])
        o_g = jax.nn.sigmoid(gates[:, 3 * hd:])
        c = f_g * cp[...] + i_g * g_g
        hn = o_g * jnp.tanh(c)
        hp[...] = hn
        cp[...] = c

        @pl.when(t == deg - 1)
        def _():
            out2 = jnp.dot(h_ref[...], ws2_ref[...],
                           preferred_element_type=jnp.float32)
            out2 = out2 + jnp.dot(hn, wn2_ref[...],
                                  preferred_element_type=jnp.float32)
            out2 = out2 + b2_ref[...]
            m = jnp.max(out2, axis=1, keepdims=True)
            e = out2 - m
            lse = jnp.log(jnp.sum(jnp.exp(e), axis=1, keepdims=True))
            o_ref[...] = e - lse

    return pl.pallas_call(
        body,
        grid=(n // bn, deg),
        in_specs=[
            pl.BlockSpec((1, bn, hd), lambda i, t: (t, i, 0)),
            pl.BlockSpec((bn, hd), lambda i, t: (i, 0)),
            pl.BlockSpec((hd, h4), lambda i, t: (0, 0)),
            pl.BlockSpec((hd, h4), lambda i, t: (0, 0)),
            pl.BlockSpec((1, h4), lambda i, t: (0, 0)),
            pl.BlockSpec((hd, c_out), lambda i, t: (0, 0)),
            pl.BlockSpec((hd, c_out), lambda i, t: (0, 0)),
            pl.BlockSpec((1, c_out), lambda i, t: (0, 0)),
        ],
        out_specs=pl.BlockSpec((bn, c_out), lambda i, t: (i, 0)),
        out_shape=jax.ShapeDtypeStruct((n, c_out), jnp.float32),
        scratch_shapes=[
            pltpu.VMEM((bn, hd), jnp.float32),
            pltpu.VMEM((bn, hd), jnp.float32),
        ],
        compiler_params=pltpu.CompilerParams(
            dimension_semantics=("parallel", "arbitrary")),
    )(seq, h, wih, whh, bg, ws2, wn2, b2.reshape(1, c_out))


def kernel(x, edge_index, W_self1, W_neigh1, b1, Wih, Whh, bih, bhh,
           W_self2, W_neigh2, b2):
    x = x.astype(jnp.float32)
    src = edge_index[0].astype(jnp.int32)
    n, d = x.shape
    e = src.shape[0]
    deg = e // n
    hd = W_self1.shape[1]

    ch_nodes = 16
    per_w = -(-n // (_NW * ch_nodes)) * ch_nodes
    npad = per_w * _NW
    pad_e = npad * deg - e
    src_pad = jnp.concatenate([src, jnp.zeros((pad_e,), jnp.int32)]) if pad_e else src
    mean_neigh = _sc_mean(x, src_pad, deg, npad, per_w, ch_nodes)[:n]

    bn = max(b for b in range(8, 2001, 8) if n % b == 0)
    h = _tc_layer1(x, mean_neigh, W_self1, W_neigh1, b1, bn)

    src_tm = src.reshape(n, deg).T.reshape(e)
    per_w_e = e // _NW
    ch = max(c for c in range(8, 401, 8) if per_w_e % c == 0)
    seq = _sc_gather(h, src_tm, per_w_e, ch).reshape(deg, n, hd)

    bn2 = max(b for b in range(8, 1001, 8) if n % b == 0)
    bg = (bih + bhh).reshape(1, Wih.shape[1])
    return _tc_lstm_out(seq, h, Wih, Whh, bg, W_self2, W_neigh2, b2, bn2)
